# baseline (speedup 1.0000x reference)
import functools

import jax
import jax.numpy as jnp
from jax import lax
from jax.experimental import pallas as pl
from jax.experimental.pallas import tpu as pltpu
from jax.experimental.pallas import tpu_sc as plsc

N = 10000
E = 320000
D = 128
R = 8
NB = 4

NOCT = 8
OCT = D // NOCT
SUB = 128
WAVE = 1
NBANK = 4
HBANK = 2
EP = 327680
NROW = EP // SUB
ROWS_PER_TILE = NROW // 16
NITER = ROWS_PER_TILE // (WAVE * NBANK)
SEGS = R * N
SEG_PER_TILE = SEGS // 16
NSH = N // 16
CH = 4
NCH = ROWS_PER_TILE // CH
ZR = 125
NZ = SEG_PER_TILE // ZR


def _sc_body(xT, xr2, pk3, wg2, out,
             stag, pk_res,
             gi0, gi1, gi2, gi3, si0, si1, si2, si3,
             wb0, wb1, wb2, wb3, xb0, xb1, xb2, xb3, zbuf, xoct, acc,
             sem_stag, sem_z, sem_wb, sem_x,
             sg0, sg1, sg2, sg3, ss0, ss1, ss2, ss3,
             sw0, sw1, sw2, sw3):
    c = lax.axis_index("c")
    s = lax.axis_index("s")
    gi = (gi0, gi1, gi2, gi3)
    si = (si0, si1, si2, si3)
    wb = (wb0, wb1, wb2, wb3)
    xb = (xb0, xb1, xb2, xb3)
    sg = (sg0, sg1, sg2, sg3)
    ss = (ss0, ss1, ss2, ss3)
    sw = (sw0, sw1, sw2, sw3)
    row0 = s * ROWS_PER_TILE
    xsh = pl.ds(s * NSH, NSH)

    @pl.loop(0, ZR)
    def _z(i):
        zbuf[i, :] = jnp.zeros((OCT,), jnp.float32)

    for z in range(NZ):
        pltpu.async_copy(
            zbuf, acc.at[pl.ds(s * SEG_PER_TILE + z * ZR, ZR)], sem_z)
    pltpu.async_copy(xT.at[c * (NOCT // 2), xsh, :], xoct.at[xsh], sem_x)

    pltpu.async_copy(pk3.at[pl.ds(row0, CH)], stag.at[pl.ds(0, CH)], sem_stag)

    @pl.loop(0, NCH)
    def _chunk(ch):
        pltpu.make_async_copy(pk3.at[pl.ds(0, CH)],
                              stag.at[pl.ds(0, CH)], sem_stag).wait()
        p = lax.rem(ch, 2) * CH

        @pl.when(ch < NCH - 1)
        def _():
            nxt = lax.rem(ch + 1, 2) * CH
            pltpu.async_copy(pk3.at[pl.ds(row0 + (ch + 1) * CH, CH)],
                             stag.at[pl.ds(nxt, CH)], sem_stag)

        for j in range(CH):
            r = ch * CH + j
            for g in range(SUB // 16):
                sl = pl.ds(g * 16, 16)
                pk_res[r, sl] = (
                    (stag[p + j, 2, sl] * N + stag[p + j, 1, sl]) * 16384
                    + stag[p + j, 0, sl])

    def _phase1(it, k, first, oct_g):
        if not first:
            for j in range(WAVE):
                pltpu.make_async_copy(xT.at[0, pl.ds(0, SUB), :],
                                      xb[k].at[j], ss[k]).wait()
        blk = it * NBANK + k
        pltpu.async_copy(wg2.at[pl.ds(row0 + blk * WAVE, WAVE)], wb[k], sw[k])
        for j in range(WAVE):
            row = blk * WAVE + j
            for g in range(SUB // 16):
                sl = pl.ds(g * 16, 16)
                pk = pk_res[row, sl]
                if k < HBANK:
                    gi[k][j, sl] = lax.bitwise_and(pk, 16383)
                else:
                    gi[k][j, sl] = lax.bitwise_and(pk, 16383) + oct_g * N
                si[k][j, sl] = lax.shift_right_logical(pk, 14)
            if k < HBANK:
                pltpu.async_copy(xoct.at[gi[k].at[j]], xb[k].at[j], sg[k])
            else:
                pltpu.async_copy(xr2.at[gi[k].at[j]], xb[k].at[j], sg[k])

    def _phase2(it, k):
        for j in range(WAVE):
            pltpu.make_async_copy(xT.at[0, pl.ds(0, SUB), :],
                                  xb[k].at[j], sg[k]).wait()
        pltpu.make_async_copy(wg2.at[pl.ds(0, WAVE)], wb[k], sw[k]).wait()
        for j in range(WAVE):

            @pl.loop(0, SUB // 16)
            def _g(g):
                wrow = wb[k][j, pl.ds(g * 16, 16)]
                for dk in range(16):
                    xb[k][j, g * 16 + dk, :] = (
                        xb[k][j, g * 16 + dk, :] * wrow[dk])

            pltpu.async_copy(xb[k].at[j], acc.at[si[k].at[j]], ss[k],
                             add=True)

    stripe = pl.ds(s * SEG_PER_TILE, SEG_PER_TILE)

    @pl.loop(0, NOCT // 2)
    def _pass(oct_local):
        oct_g = c * (NOCT // 2) + oct_local

        @pl.when(oct_local > 0)
        def _():
            pltpu.async_copy(xT.at[oct_g, xsh, :], xoct.at[xsh], sem_x)
            pltpu.make_async_copy(acc.at[stripe], out.at[stripe, 0, :],
                                  sem_wb).wait()
            for z in range(NZ):
                pltpu.async_copy(
                    zbuf, acc.at[pl.ds(s * SEG_PER_TILE + z * ZR, ZR)],
                    sem_z)

        for z in range(NZ):
            pltpu.make_async_copy(xT.at[0, pl.ds(0, ZR), :], zbuf,
                                  sem_z).wait()
        pltpu.make_async_copy(xT.at[0, xsh, :], xoct.at[xsh], sem_x).wait()
        plsc.subcore_barrier()

        for k in range(NBANK):
            _phase1(0, k, True, oct_g)
        for k in range(NBANK):
            _phase2(0, k)

        @pl.loop(1, NITER)
        def _it(it):
            for k in range(NBANK):
                _phase1(it, k, False, oct_g)
            for k in range(NBANK):
                _phase2(it, k)

        for k in range(NBANK):
            for j in range(WAVE):
                pltpu.make_async_copy(xT.at[0, pl.ds(0, SUB), :],
                                      xb[k].at[j], ss[k]).wait()
        plsc.subcore_barrier()
        pltpu.async_copy(acc.at[stripe], out.at[stripe, oct_g, :], sem_wb)

    pltpu.make_async_copy(acc.at[stripe], out.at[stripe, 0, :],
                          sem_wb).wait()


@jax.jit
def _sc_aggregate(xT, xr2, pk3, wg2):
    mesh = plsc.VectorSubcoreMesh(core_axis_name="c", subcore_axis_name="s")
    kern = pl.kernel(
        _sc_body,
        out_type=jax.ShapeDtypeStruct((SEGS, NOCT, OCT), jnp.float32),
        mesh=mesh,
        scratch_types=(
            [pltpu.VMEM((2 * CH, 3, SUB), jnp.int32)]
            + [pltpu.VMEM((ROWS_PER_TILE, SUB), jnp.int32)]
            + [pltpu.VMEM((WAVE, SUB), jnp.int32) for _ in range(8)]
            + [pltpu.VMEM((WAVE, SUB), jnp.float32) for _ in range(4)]
            + [pltpu.VMEM((WAVE, SUB, OCT), jnp.float32) for _ in range(4)]
            + [pltpu.VMEM((ZR, OCT), jnp.float32)]
            + [pltpu.VMEM_SHARED((N, OCT), jnp.float32)]
            + [pltpu.VMEM_SHARED((SEGS, OCT), jnp.float32)]
            + [pltpu.SemaphoreType.DMA for _ in range(16)]
        ),
        compiler_params=pltpu.CompilerParams(use_tc_tiling_on_sc=False),
    )
    return kern(xT, xr2, pk3, wg2)


BN = 1000


def _tc_body(wc_ref, agg_ref, wb_ref, o_ref):
    acc = jnp.zeros((BN, D), jnp.float32)
    for b in range(NB):
        ab = jnp.zeros((BN, D), jnp.float32)
        for r in range(R):
            ab = ab + wc_ref[r, b] * agg_ref[r]
        acc = acc + jnp.dot(ab, wb_ref[b], preferred_element_type=jnp.float32)
    o_ref[...] = acc


@jax.jit
def _tc_apply(agg3, W_bases, w_comp):
    return pl.pallas_call(
        _tc_body,
        grid=(N // BN,),
        in_specs=[
            pl.BlockSpec(memory_space=pltpu.SMEM),
            pl.BlockSpec((R, BN, D), lambda i: (0, i, 0)),
            pl.BlockSpec((NB, D, D), lambda i: (0, 0, 0)),
        ],
        out_specs=pl.BlockSpec((BN, D), lambda i: (i, 0)),
        out_shape=jax.ShapeDtypeStruct((N, D), jnp.float32),
    )(w_comp, agg3, W_bases)


@jax.jit
def _impl(x, edge_index, edge_type, edge_weight, W_bases, w_comp):
    pad = EP - E
    src = jnp.concatenate([edge_index[0], jnp.zeros((pad,), jnp.int32)])
    dst = jnp.concatenate([edge_index[1], jnp.zeros((pad,), jnp.int32)])
    typ = jnp.concatenate([edge_type, jnp.zeros((pad,), jnp.int32)])
    wgt = jnp.concatenate([edge_weight, jnp.zeros((pad,), jnp.float32)])
    pk3 = jnp.stack(
        [src.reshape(NROW, SUB), dst.reshape(NROW, SUB),
         typ.reshape(NROW, SUB)], axis=1)
    wg2 = wgt.reshape(NROW, SUB)
    xT = x.reshape(N, NOCT, OCT).transpose(1, 0, 2)
    xr2 = xT.reshape(NOCT * N, OCT)
    agg = _sc_aggregate(xT, xr2, pk3, wg2)
    agg3 = agg.reshape(R, N, D)
    return _tc_apply(agg3, W_bases, w_comp)


def kernel(x, edge_index, edge_type, edge_weight, W_bases, w_comp):
    return _impl(x, edge_index, edge_type, edge_weight, W_bases, w_comp)


# speedup vs baseline: 1.0744x; 1.0744x over previous
import functools

import jax
import jax.numpy as jnp
from jax import lax
from jax.experimental import pallas as pl
from jax.experimental.pallas import tpu as pltpu
from jax.experimental.pallas import tpu_sc as plsc

N = 10000
E = 320000
D = 128
R = 8
NB = 4

NOCT = 8
OCT = D // NOCT
SUB = 128
WAVE = 1
NBANK = 4
HBANK = 3
EP = 327680
NROW = EP // SUB
ROWS_PER_TILE = NROW // 16
NITER = ROWS_PER_TILE // (WAVE * NBANK)
SEGS = R * N
SEG_PER_TILE = SEGS // 16
NSH = N // 16
CH = 4
NCH = ROWS_PER_TILE // CH
ZR = 125
NZ = SEG_PER_TILE // ZR


def _sc_body(xT, xr2, pk3, wg2, out,
             stag, pk_res,
             gi0, gi1, gi2, gi3, si0, si1, si2, si3,
             wb0, wb1, wb2, wb3, xb0, xb1, xb2, xb3, zbuf, xoct, acc,
             sem_stag, sem_z, sem_wb, sem_x,
             sg0, sg1, sg2, sg3, ss0, ss1, ss2, ss3,
             sw0, sw1, sw2, sw3):
    c = lax.axis_index("c")
    s = lax.axis_index("s")
    gi = (gi0, gi1, gi2, gi3)
    si = (si0, si1, si2, si3)
    wb = (wb0, wb1, wb2, wb3)
    xb = (xb0, xb1, xb2, xb3)
    sg = (sg0, sg1, sg2, sg3)
    ss = (ss0, ss1, ss2, ss3)
    sw = (sw0, sw1, sw2, sw3)
    row0 = s * ROWS_PER_TILE
    xsh = pl.ds(s * NSH, NSH)

    @pl.loop(0, ZR)
    def _z(i):
        zbuf[i, :] = jnp.zeros((OCT,), jnp.float32)

    for z in range(NZ):
        pltpu.async_copy(
            zbuf, acc.at[pl.ds(s * SEG_PER_TILE + z * ZR, ZR)], sem_z)
    pltpu.async_copy(xT.at[c * (NOCT // 2), xsh, :], xoct.at[xsh], sem_x)

    pltpu.async_copy(pk3.at[pl.ds(row0, CH)], stag.at[pl.ds(0, CH)], sem_stag)

    @pl.loop(0, NCH)
    def _chunk(ch):
        pltpu.make_async_copy(pk3.at[pl.ds(0, CH)],
                              stag.at[pl.ds(0, CH)], sem_stag).wait()
        p = lax.rem(ch, 2) * CH

        @pl.when(ch < NCH - 1)
        def _():
            nxt = lax.rem(ch + 1, 2) * CH
            pltpu.async_copy(pk3.at[pl.ds(row0 + (ch + 1) * CH, CH)],
                             stag.at[pl.ds(nxt, CH)], sem_stag)

        for j in range(CH):
            r = ch * CH + j
            for g in range(SUB // 16):
                sl = pl.ds(g * 16, 16)
                pk_res[r, sl] = (
                    (stag[p + j, 2, sl] * N + stag[p + j, 1, sl]) * 16384
                    + stag[p + j, 0, sl])

    def _phase1(it, k, first, oct_g):
        if not first:
            for j in range(WAVE):
                pltpu.make_async_copy(xT.at[0, pl.ds(0, SUB), :],
                                      xb[k].at[j], ss[k]).wait()
        blk = it * NBANK + k
        pltpu.async_copy(wg2.at[pl.ds(row0 + blk * WAVE, WAVE)], wb[k], sw[k])
        for j in range(WAVE):
            row = blk * WAVE + j
            for g in range(SUB // 16):
                sl = pl.ds(g * 16, 16)
                pk = pk_res[row, sl]
                if k < HBANK:
                    gi[k][j, sl] = lax.bitwise_and(pk, 16383)
                else:
                    gi[k][j, sl] = lax.bitwise_and(pk, 16383) + oct_g * N
                si[k][j, sl] = lax.shift_right_logical(pk, 14)
            if k < HBANK:
                pltpu.async_copy(xoct.at[gi[k].at[j]], xb[k].at[j], sg[k])
            else:
                pltpu.async_copy(xr2.at[gi[k].at[j]], xb[k].at[j], sg[k])

    def _phase2(it, k):
        for j in range(WAVE):
            pltpu.make_async_copy(xT.at[0, pl.ds(0, SUB), :],
                                  xb[k].at[j], sg[k]).wait()
        pltpu.make_async_copy(wg2.at[pl.ds(0, WAVE)], wb[k], sw[k]).wait()
        for j in range(WAVE):

            @pl.loop(0, SUB // 16)
            def _g(g):
                wrow = wb[k][j, pl.ds(g * 16, 16)]
                for dk in range(16):
                    xb[k][j, g * 16 + dk, :] = (
                        xb[k][j, g * 16 + dk, :] * wrow[dk])

            pltpu.async_copy(xb[k].at[j], acc.at[si[k].at[j]], ss[k],
                             add=True)

    stripe = pl.ds(s * SEG_PER_TILE, SEG_PER_TILE)

    @pl.loop(0, NOCT // 2)
    def _pass(oct_local):
        oct_g = c * (NOCT // 2) + oct_local

        @pl.when(oct_local > 0)
        def _():
            pltpu.async_copy(xT.at[oct_g, xsh, :], xoct.at[xsh], sem_x)
            pltpu.make_async_copy(acc.at[stripe], out.at[stripe, 0, :],
                                  sem_wb).wait()
            for z in range(NZ):
                pltpu.async_copy(
                    zbuf, acc.at[pl.ds(s * SEG_PER_TILE + z * ZR, ZR)],
                    sem_z)

        for z in range(NZ):
            pltpu.make_async_copy(xT.at[0, pl.ds(0, ZR), :], zbuf,
                                  sem_z).wait()
        pltpu.make_async_copy(xT.at[0, xsh, :], xoct.at[xsh], sem_x).wait()
        plsc.subcore_barrier()

        for k in range(NBANK):
            _phase1(0, k, True, oct_g)
        for k in range(NBANK):
            _phase2(0, k)

        @pl.loop(1, NITER)
        def _it(it):
            for k in range(NBANK):
                _phase1(it, k, False, oct_g)
            for k in range(NBANK):
                _phase2(it, k)

        for k in range(NBANK):
            for j in range(WAVE):
                pltpu.make_async_copy(xT.at[0, pl.ds(0, SUB), :],
                                      xb[k].at[j], ss[k]).wait()
        plsc.subcore_barrier()
        pltpu.async_copy(acc.at[stripe], out.at[stripe, oct_g, :], sem_wb)

    pltpu.make_async_copy(acc.at[stripe], out.at[stripe, 0, :],
                          sem_wb).wait()


@jax.jit
def _sc_aggregate(xT, xr2, pk3, wg2):
    mesh = plsc.VectorSubcoreMesh(core_axis_name="c", subcore_axis_name="s")
    kern = pl.kernel(
        _sc_body,
        out_type=jax.ShapeDtypeStruct((SEGS, NOCT, OCT), jnp.float32),
        mesh=mesh,
        scratch_types=(
            [pltpu.VMEM((2 * CH, 3, SUB), jnp.int32)]
            + [pltpu.VMEM((ROWS_PER_TILE, SUB), jnp.int32)]
            + [pltpu.VMEM((WAVE, SUB), jnp.int32) for _ in range(8)]
            + [pltpu.VMEM((WAVE, SUB), jnp.float32) for _ in range(4)]
            + [pltpu.VMEM((WAVE, SUB, OCT), jnp.float32) for _ in range(4)]
            + [pltpu.VMEM((ZR, OCT), jnp.float32)]
            + [pltpu.VMEM_SHARED((N, OCT), jnp.float32)]
            + [pltpu.VMEM_SHARED((SEGS, OCT), jnp.float32)]
            + [pltpu.SemaphoreType.DMA for _ in range(16)]
        ),
        compiler_params=pltpu.CompilerParams(use_tc_tiling_on_sc=False),
    )
    return kern(xT, xr2, pk3, wg2)


BN = 1000


def _tc_body(wc_ref, agg_ref, wb_ref, o_ref):
    acc = jnp.zeros((BN, D), jnp.float32)
    for b in range(NB):
        ab = jnp.zeros((BN, D), jnp.float32)
        for r in range(R):
            ab = ab + wc_ref[r, b] * agg_ref[r]
        acc = acc + jnp.dot(ab, wb_ref[b], preferred_element_type=jnp.float32)
    o_ref[...] = acc


@jax.jit
def _tc_apply(agg3, W_bases, w_comp):
    return pl.pallas_call(
        _tc_body,
        grid=(N // BN,),
        in_specs=[
            pl.BlockSpec(memory_space=pltpu.SMEM),
            pl.BlockSpec((R, BN, D), lambda i: (0, i, 0)),
            pl.BlockSpec((NB, D, D), lambda i: (0, 0, 0)),
        ],
        out_specs=pl.BlockSpec((BN, D), lambda i: (i, 0)),
        out_shape=jax.ShapeDtypeStruct((N, D), jnp.float32),
    )(w_comp, agg3, W_bases)


@jax.jit
def _impl(x, edge_index, edge_type, edge_weight, W_bases, w_comp):
    pad = EP - E
    src = jnp.concatenate([edge_index[0], jnp.zeros((pad,), jnp.int32)])
    dst = jnp.concatenate([edge_index[1], jnp.zeros((pad,), jnp.int32)])
    typ = jnp.concatenate([edge_type, jnp.zeros((pad,), jnp.int32)])
    wgt = jnp.concatenate([edge_weight, jnp.zeros((pad,), jnp.float32)])
    pk3 = jnp.stack(
        [src.reshape(NROW, SUB), dst.reshape(NROW, SUB),
         typ.reshape(NROW, SUB)], axis=1)
    wg2 = wgt.reshape(NROW, SUB)
    xT = x.reshape(N, NOCT, OCT).transpose(1, 0, 2)
    xr2 = xT.reshape(NOCT * N, OCT)
    agg = _sc_aggregate(xT, xr2, pk3, wg2)
    agg3 = agg.reshape(R, N, D)
    return _tc_apply(agg3, W_bases, w_comp)


def kernel(x, edge_index, edge_type, edge_weight, W_bases, w_comp):
    return _impl(x, edge_index, edge_type, edge_weight, W_bases, w_comp)


# speedup vs baseline: 1.1735x; 1.0922x over previous
import functools

import jax
import jax.numpy as jnp
from jax import lax
from jax.experimental import pallas as pl
from jax.experimental.pallas import tpu as pltpu
from jax.experimental.pallas import tpu_sc as plsc

N = 10000
E = 320000
D = 128
R = 8
NB = 4

NOCT = 8
OCT = D // NOCT
SUB = 128
WAVE = 1
NBANK = 4
EP = 327680
NROW = EP // SUB
ROWS_PER_TILE = NROW // 16
NITER = ROWS_PER_TILE // (WAVE * NBANK)
SEGS = R * N
SEG_PER_TILE = SEGS // 16
NSH = N // 16
CH = 4
NCH = ROWS_PER_TILE // CH
ZR = 125
NZ = SEG_PER_TILE // ZR


def _sc_body(xT, pk3, wg2, out,
             stag, pk_res,
             gi0, gi1, gi2, gi3, si0, si1, si2, si3,
             wb0, wb1, wb2, wb3, xb0, xb1, xb2, xb3, zbuf, xoct, acc,
             sem_stag, sem_z, sem_wb, sem_x,
             sg0, sg1, sg2, sg3, ss0, ss1, ss2, ss3,
             sw0, sw1, sw2, sw3):
    c = lax.axis_index("c")
    s = lax.axis_index("s")
    gi = (gi0, gi1, gi2, gi3)
    si = (si0, si1, si2, si3)
    wb = (wb0, wb1, wb2, wb3)
    xb = (xb0, xb1, xb2, xb3)
    sg = (sg0, sg1, sg2, sg3)
    ss = (ss0, ss1, ss2, ss3)
    sw = (sw0, sw1, sw2, sw3)
    row0 = s * ROWS_PER_TILE
    xsh = pl.ds(s * NSH, NSH)

    @pl.loop(0, ZR)
    def _z(i):
        zbuf[i, :] = jnp.zeros((OCT,), jnp.float32)

    for z in range(NZ):
        pltpu.async_copy(
            zbuf, acc.at[pl.ds(s * SEG_PER_TILE + z * ZR, ZR)], sem_z)
    pltpu.async_copy(xT.at[c * (NOCT // 2), xsh, :], xoct.at[xsh], sem_x)

    pltpu.async_copy(pk3.at[pl.ds(row0, CH)], stag.at[pl.ds(0, CH)], sem_stag)

    @pl.loop(0, NCH)
    def _chunk(ch):
        pltpu.make_async_copy(pk3.at[pl.ds(0, CH)],
                              stag.at[pl.ds(0, CH)], sem_stag).wait()
        p = lax.rem(ch, 2) * CH

        @pl.when(ch < NCH - 1)
        def _():
            nxt = lax.rem(ch + 1, 2) * CH
            pltpu.async_copy(pk3.at[pl.ds(row0 + (ch + 1) * CH, CH)],
                             stag.at[pl.ds(nxt, CH)], sem_stag)

        for j in range(CH):
            r = ch * CH + j
            for g in range(SUB // 16):
                sl = pl.ds(g * 16, 16)
                pk_res[r, sl] = (
                    (stag[p + j, 2, sl] * N + stag[p + j, 1, sl]) * 16384
                    + stag[p + j, 0, sl])

    def _phase1(it, k, first):
        if not first:
            for j in range(WAVE):
                pltpu.make_async_copy(xT.at[0, pl.ds(0, SUB), :],
                                      xb[k].at[j], ss[k]).wait()
        blk = it * NBANK + k
        pltpu.async_copy(wg2.at[pl.ds(row0 + blk * WAVE, WAVE)], wb[k], sw[k])
        for j in range(WAVE):
            row = blk * WAVE + j
            for g in range(SUB // 16):
                sl = pl.ds(g * 16, 16)
                pk = pk_res[row, sl]
                gi[k][j, sl] = lax.bitwise_and(pk, 16383)
                si[k][j, sl] = lax.shift_right_logical(pk, 14)
            pltpu.async_copy(xoct.at[gi[k].at[j]], xb[k].at[j], sg[k])

    def _phase2(it, k):
        for j in range(WAVE):
            pltpu.make_async_copy(xT.at[0, pl.ds(0, SUB), :],
                                  xb[k].at[j], sg[k]).wait()
        pltpu.make_async_copy(wg2.at[pl.ds(0, WAVE)], wb[k], sw[k]).wait()
        for j in range(WAVE):

            @pl.loop(0, SUB // 16)
            def _g(g):
                wrow = wb[k][j, pl.ds(g * 16, 16)]
                for dk in range(16):
                    xb[k][j, g * 16 + dk, :] = (
                        xb[k][j, g * 16 + dk, :] * wrow[dk])

            pltpu.async_copy(xb[k].at[j], acc.at[si[k].at[j]], ss[k],
                             add=True)

    stripe = pl.ds(s * SEG_PER_TILE, SEG_PER_TILE)

    @pl.loop(0, NOCT // 2)
    def _pass(oct_local):
        oct_g = c * (NOCT // 2) + oct_local

        @pl.when(oct_local > 0)
        def _():
            pltpu.async_copy(xT.at[oct_g, xsh, :], xoct.at[xsh], sem_x)
            pltpu.make_async_copy(acc.at[stripe], out.at[stripe, 0, :],
                                  sem_wb).wait()
            for z in range(NZ):
                pltpu.async_copy(
                    zbuf, acc.at[pl.ds(s * SEG_PER_TILE + z * ZR, ZR)],
                    sem_z)

        for z in range(NZ):
            pltpu.make_async_copy(xT.at[0, pl.ds(0, ZR), :], zbuf,
                                  sem_z).wait()
        pltpu.make_async_copy(xT.at[0, xsh, :], xoct.at[xsh], sem_x).wait()
        plsc.subcore_barrier()

        for k in range(NBANK):
            _phase1(0, k, True)
        for k in range(NBANK):
            _phase2(0, k)

        @pl.loop(1, NITER)
        def _it(it):
            for k in range(NBANK):
                _phase1(it, k, False)
            for k in range(NBANK):
                _phase2(it, k)

        for k in range(NBANK):
            for j in range(WAVE):
                pltpu.make_async_copy(xT.at[0, pl.ds(0, SUB), :],
                                      xb[k].at[j], ss[k]).wait()
        plsc.subcore_barrier()
        pltpu.async_copy(acc.at[stripe], out.at[stripe, oct_g, :], sem_wb)

    pltpu.make_async_copy(acc.at[stripe], out.at[stripe, 0, :],
                          sem_wb).wait()


@jax.jit
def _sc_aggregate(xT, pk3, wg2):
    mesh = plsc.VectorSubcoreMesh(core_axis_name="c", subcore_axis_name="s")
    kern = pl.kernel(
        _sc_body,
        out_type=jax.ShapeDtypeStruct((SEGS, NOCT, OCT), jnp.float32),
        mesh=mesh,
        scratch_types=(
            [pltpu.VMEM((2 * CH, 3, SUB), jnp.int32)]
            + [pltpu.VMEM((ROWS_PER_TILE, SUB), jnp.int32)]
            + [pltpu.VMEM((WAVE, SUB), jnp.int32) for _ in range(8)]
            + [pltpu.VMEM((WAVE, SUB), jnp.float32) for _ in range(4)]
            + [pltpu.VMEM((WAVE, SUB, OCT), jnp.float32) for _ in range(4)]
            + [pltpu.VMEM((ZR, OCT), jnp.float32)]
            + [pltpu.VMEM_SHARED((N, OCT), jnp.float32)]
            + [pltpu.VMEM_SHARED((SEGS, OCT), jnp.float32)]
            + [pltpu.SemaphoreType.DMA for _ in range(16)]
        ),
        compiler_params=pltpu.CompilerParams(use_tc_tiling_on_sc=False),
    )
    return kern(xT, pk3, wg2)


BN = 1000


def _tc_body(wc_ref, agg_ref, wb_ref, o_ref):
    acc = jnp.zeros((BN, D), jnp.float32)
    for b in range(NB):
        ab = jnp.zeros((BN, D), jnp.float32)
        for r in range(R):
            ab = ab + wc_ref[r, b] * agg_ref[r]
        acc = acc + jnp.dot(ab, wb_ref[b], preferred_element_type=jnp.float32)
    o_ref[...] = acc


@jax.jit
def _tc_apply(agg3, W_bases, w_comp):
    return pl.pallas_call(
        _tc_body,
        grid=(N // BN,),
        in_specs=[
            pl.BlockSpec(memory_space=pltpu.SMEM),
            pl.BlockSpec((R, BN, D), lambda i: (0, i, 0)),
            pl.BlockSpec((NB, D, D), lambda i: (0, 0, 0)),
        ],
        out_specs=pl.BlockSpec((BN, D), lambda i: (i, 0)),
        out_shape=jax.ShapeDtypeStruct((N, D), jnp.float32),
    )(w_comp, agg3, W_bases)


@jax.jit
def _impl(x, edge_index, edge_type, edge_weight, W_bases, w_comp):
    pad = EP - E
    src = jnp.concatenate([edge_index[0], jnp.zeros((pad,), jnp.int32)])
    dst = jnp.concatenate([edge_index[1], jnp.zeros((pad,), jnp.int32)])
    typ = jnp.concatenate([edge_type, jnp.zeros((pad,), jnp.int32)])
    wgt = jnp.concatenate([edge_weight, jnp.zeros((pad,), jnp.float32)])
    pk3 = jnp.stack(
        [src.reshape(NROW, SUB), dst.reshape(NROW, SUB),
         typ.reshape(NROW, SUB)], axis=1)
    wg2 = wgt.reshape(NROW, SUB)
    xT = x.reshape(N, NOCT, OCT).transpose(1, 0, 2)
    agg = _sc_aggregate(xT, pk3, wg2)
    agg3 = agg.reshape(R, N, D)
    return _tc_apply(agg3, W_bases, w_comp)


def kernel(x, edge_index, edge_type, edge_weight, W_bases, w_comp):
    return _impl(x, edge_index, edge_type, edge_weight, W_bases, w_comp)


# speedup vs baseline: 1.1875x; 1.0120x over previous
import functools

import jax
import jax.numpy as jnp
from jax import lax
from jax.experimental import pallas as pl
from jax.experimental.pallas import tpu as pltpu
from jax.experimental.pallas import tpu_sc as plsc

N = 10000
E = 320000
D = 128
R = 8
NB = 4

NOCT = 8
OCT = D // NOCT
SUB = 128
WAVE = 1
NBANK = 4
EP = 327680
NROW = EP // SUB
ROWS_PER_TILE = NROW // 16
NITER = ROWS_PER_TILE // (WAVE * NBANK)
SEGS = R * N
SEG_PER_TILE = SEGS // 16
NSH = N // 16
CH = 4
NCH = ROWS_PER_TILE // CH
ZR = 125
NZ = SEG_PER_TILE // ZR


def _sc_body(xT, src2, dst2, typ2, wg2, out,
             stag, pk_res,
             gi0, gi1, gi2, gi3, si0, si1, si2, si3,
             wb0, wb1, wb2, wb3, xb0, xb1, xb2, xb3, zbuf, xoct, acc,
             sem_stag, sem_z, sem_wb, sem_x,
             sg0, sg1, sg2, sg3, ss0, ss1, ss2, ss3,
             sw0, sw1, sw2, sw3):
    c = lax.axis_index("c")
    s = lax.axis_index("s")
    gi = (gi0, gi1, gi2, gi3)
    si = (si0, si1, si2, si3)
    wb = (wb0, wb1, wb2, wb3)
    xb = (xb0, xb1, xb2, xb3)
    sg = (sg0, sg1, sg2, sg3)
    ss = (ss0, ss1, ss2, ss3)
    sw = (sw0, sw1, sw2, sw3)
    row0 = s * ROWS_PER_TILE
    xsh = pl.ds(s * NSH, NSH)

    @pl.loop(0, ZR)
    def _z(i):
        zbuf[i, :] = jnp.zeros((OCT,), jnp.float32)

    for z in range(NZ):
        pltpu.async_copy(
            zbuf, acc.at[pl.ds(s * SEG_PER_TILE + z * ZR, ZR)], sem_z)
    pltpu.async_copy(xT.at[c * (NOCT // 2), xsh, :], xoct.at[xsh], sem_x)

    def _load_chunk(gr, p):
        pltpu.async_copy(src2.at[pl.ds(gr, CH)], stag.at[pl.ds(p, CH), 0, :],
                         sem_stag)
        pltpu.async_copy(dst2.at[pl.ds(gr, CH)], stag.at[pl.ds(p, CH), 1, :],
                         sem_stag)
        pltpu.async_copy(typ2.at[pl.ds(gr, CH)], stag.at[pl.ds(p, CH), 2, :],
                         sem_stag)

    _load_chunk(row0, 0)

    @pl.loop(0, NCH)
    def _chunk(ch):
        for _ in range(3):
            pltpu.make_async_copy(src2.at[pl.ds(0, CH)],
                                  stag.at[pl.ds(0, CH), 0, :],
                                  sem_stag).wait()
        p = lax.rem(ch, 2) * CH

        @pl.when(ch < NCH - 1)
        def _():
            _load_chunk(row0 + (ch + 1) * CH, lax.rem(ch + 1, 2) * CH)

        for j in range(CH):
            r = ch * CH + j
            for g in range(SUB // 16):
                sl = pl.ds(g * 16, 16)
                pk_res[r, sl] = (
                    (stag[p + j, 2, sl] * N + stag[p + j, 1, sl]) * 16384
                    + stag[p + j, 0, sl])

    def _phase1(it, k, first):
        if not first:
            for j in range(WAVE):
                pltpu.make_async_copy(xT.at[0, pl.ds(0, SUB), :],
                                      xb[k].at[j], ss[k]).wait()
        blk = it * NBANK + k
        pltpu.async_copy(wg2.at[pl.ds(row0 + blk * WAVE, WAVE)], wb[k], sw[k])
        for j in range(WAVE):
            row = blk * WAVE + j
            for g in range(SUB // 16):
                sl = pl.ds(g * 16, 16)
                pk = pk_res[row, sl]
                gi[k][j, sl] = lax.bitwise_and(pk, 16383)
                si[k][j, sl] = lax.shift_right_logical(pk, 14)
            pltpu.async_copy(xoct.at[gi[k].at[j]], xb[k].at[j], sg[k])

    def _phase2(it, k):
        for j in range(WAVE):
            pltpu.make_async_copy(xT.at[0, pl.ds(0, SUB), :],
                                  xb[k].at[j], sg[k]).wait()
        pltpu.make_async_copy(wg2.at[pl.ds(0, WAVE)], wb[k], sw[k]).wait()
        for j in range(WAVE):

            @pl.loop(0, SUB // 16)
            def _g(g):
                wrow = wb[k][j, pl.ds(g * 16, 16)]
                for dk in range(16):
                    xb[k][j, g * 16 + dk, :] = (
                        xb[k][j, g * 16 + dk, :] * wrow[dk])

            pltpu.async_copy(xb[k].at[j], acc.at[si[k].at[j]], ss[k],
                             add=True)

    stripe = pl.ds(s * SEG_PER_TILE, SEG_PER_TILE)

    @pl.loop(0, NOCT // 2)
    def _pass(oct_local):
        oct_g = c * (NOCT // 2) + oct_local

        @pl.when(oct_local > 0)
        def _():
            pltpu.async_copy(xT.at[oct_g, xsh, :], xoct.at[xsh], sem_x)
            pltpu.make_async_copy(acc.at[stripe], out.at[stripe, 0, :],
                                  sem_wb).wait()
            for z in range(NZ):
                pltpu.async_copy(
                    zbuf, acc.at[pl.ds(s * SEG_PER_TILE + z * ZR, ZR)],
                    sem_z)

        for z in range(NZ):
            pltpu.make_async_copy(xT.at[0, pl.ds(0, ZR), :], zbuf,
                                  sem_z).wait()
        pltpu.make_async_copy(xT.at[0, xsh, :], xoct.at[xsh], sem_x).wait()
        plsc.subcore_barrier()

        for k in range(NBANK):
            _phase1(0, k, True)
        for k in range(NBANK):
            _phase2(0, k)

        @pl.loop(1, NITER)
        def _it(it):
            for k in range(NBANK):
                _phase1(it, k, False)
            for k in range(NBANK):
                _phase2(it, k)

        for k in range(NBANK):
            for j in range(WAVE):
                pltpu.make_async_copy(xT.at[0, pl.ds(0, SUB), :],
                                      xb[k].at[j], ss[k]).wait()
        plsc.subcore_barrier()
        pltpu.async_copy(acc.at[stripe], out.at[stripe, oct_g, :], sem_wb)

    pltpu.make_async_copy(acc.at[stripe], out.at[stripe, 0, :],
                          sem_wb).wait()


@jax.jit
def _sc_aggregate(xT, src2, dst2, typ2, wg2):
    mesh = plsc.VectorSubcoreMesh(core_axis_name="c", subcore_axis_name="s")
    kern = pl.kernel(
        _sc_body,
        out_type=jax.ShapeDtypeStruct((SEGS, NOCT, OCT), jnp.float32),
        mesh=mesh,
        scratch_types=(
            [pltpu.VMEM((2 * CH, 3, SUB), jnp.int32)]
            + [pltpu.VMEM((ROWS_PER_TILE, SUB), jnp.int32)]
            + [pltpu.VMEM((WAVE, SUB), jnp.int32) for _ in range(8)]
            + [pltpu.VMEM((WAVE, SUB), jnp.float32) for _ in range(4)]
            + [pltpu.VMEM((WAVE, SUB, OCT), jnp.float32) for _ in range(4)]
            + [pltpu.VMEM((ZR, OCT), jnp.float32)]
            + [pltpu.VMEM_SHARED((N, OCT), jnp.float32)]
            + [pltpu.VMEM_SHARED((SEGS, OCT), jnp.float32)]
            + [pltpu.SemaphoreType.DMA for _ in range(16)]
        ),
        compiler_params=pltpu.CompilerParams(use_tc_tiling_on_sc=False),
    )
    return kern(xT, src2, dst2, typ2, wg2)


BN = 1000


def _tc_body(wc_ref, agg_ref, wb_ref, o_ref):
    acc = jnp.zeros((BN, D), jnp.float32)
    for b in range(NB):
        ab = jnp.zeros((BN, D), jnp.float32)
        for r in range(R):
            ab = ab + wc_ref[r, b] * agg_ref[r]
        acc = acc + jnp.dot(ab, wb_ref[b], preferred_element_type=jnp.float32)
    o_ref[...] = acc


@jax.jit
def _tc_apply(agg3, W_bases, w_comp):
    return pl.pallas_call(
        _tc_body,
        grid=(N // BN,),
        in_specs=[
            pl.BlockSpec(memory_space=pltpu.SMEM),
            pl.BlockSpec((R, BN, D), lambda i: (0, i, 0)),
            pl.BlockSpec((NB, D, D), lambda i: (0, 0, 0)),
        ],
        out_specs=pl.BlockSpec((BN, D), lambda i: (i, 0)),
        out_shape=jax.ShapeDtypeStruct((N, D), jnp.float32),
    )(w_comp, agg3, W_bases)


@jax.jit
def _impl(x, edge_index, edge_type, edge_weight, W_bases, w_comp):
    pad = EP - E
    src = jnp.concatenate([edge_index[0], jnp.zeros((pad,), jnp.int32)])
    dst = jnp.concatenate([edge_index[1], jnp.zeros((pad,), jnp.int32)])
    typ = jnp.concatenate([edge_type, jnp.zeros((pad,), jnp.int32)])
    wgt = jnp.concatenate([edge_weight, jnp.zeros((pad,), jnp.float32)])
    wg2 = wgt.reshape(NROW, SUB)
    xT = x.reshape(N, NOCT, OCT).transpose(1, 0, 2)
    agg = _sc_aggregate(xT, src.reshape(NROW, SUB), dst.reshape(NROW, SUB),
                        typ.reshape(NROW, SUB), wg2)
    agg3 = agg.reshape(R, N, D)
    return _tc_apply(agg3, W_bases, w_comp)


def kernel(x, edge_index, edge_type, edge_weight, W_bases, w_comp):
    return _impl(x, edge_index, edge_type, edge_weight, W_bases, w_comp)
